# baseline (speedup 1.0000x reference)
import jax
import jax.numpy as jnp
import numpy as np
from jax.experimental import pallas as pl

B = 8
N = 128
P = N * N
ATOM_RAW = 64
TYPE_OUT = 25
ATOM_OUT = 64
BOND_IN = 16
BOND_OUT = 16

RB = P * BOND_IN // 128
RCHUNK = 256
NSTEPS = RB // RCHUNK


def _softplus(x):
    return jnp.maximum(x, 0.0) + jnp.log1p(jnp.exp(-jnp.abs(x)))


def _lane_expand_mat():
    jj = jax.lax.broadcasted_iota(jnp.int32, (8, 128), 0)
    l = jax.lax.broadcasted_iota(jnp.int32, (8, 128), 1)
    return (jj == l // 16).astype(jnp.float32)


def _atom_body(atom_ref, adj_ref, bond2_ref, adjS_ref,
               WA_ref, bA_ref, WB_ref, bB_ref,
               Ws_ref, Wn_ref, Wab_ref, ba_ref, Wemb_ref,
               atom_out_ref, uv_out_ref):
    a = atom_ref[0]
    adj = adj_ref[0]
    bond2 = bond2_ref[0]
    adjS = adjS_ref[0]

    h = N // 2
    t0 = jnp.tanh(jnp.dot(a[:h], WA_ref[...],
                          preferred_element_type=jnp.float32) + bA_ref[...])
    t1 = jnp.tanh(jnp.dot(a[h:], WB_ref[...],
                          preferred_element_type=jnp.float32) + bB_ref[...])
    atom_t = jnp.concatenate([t0, t1], axis=0)
    nbr = jnp.dot(adj, atom_t, preferred_element_type=jnp.float32)

    adjx = jnp.dot(adjS, _lane_expand_mat(),
                   preferred_element_type=jnp.float32)
    prod = bond2 * adjx
    lsel = jax.lax.broadcasted_iota(jnp.int32, (128, BOND_IN), 0)
    dsel = jax.lax.broadcasted_iota(jnp.int32, (128, BOND_IN), 1)
    msel = (lsel % BOND_IN == dsel).astype(jnp.float32)
    s1 = jnp.dot(prod, msel, preferred_element_type=jnp.float32)
    bond_agg = jnp.sum(s1.reshape(N, BOND_IN, BOND_IN), axis=1)

    au = _softplus(jnp.dot(atom_t, Ws_ref[...], preferred_element_type=jnp.float32)
                   + jnp.dot(nbr, Wn_ref[...], preferred_element_type=jnp.float32)
                   + jnp.dot(bond_agg, Wab_ref[...], preferred_element_type=jnp.float32)
                   + ba_ref[...])
    atom_out_ref[0] = au
    Wemb = Wemb_ref[...]
    uv_out_ref[0] = jnp.concatenate(
        [jnp.dot(au, Wemb[:ATOM_OUT], preferred_element_type=jnp.float32),
         jnp.dot(au, Wemb[ATOM_OUT:], preferred_element_type=jnp.float32)],
        axis=1)


def _bond_body(t0_ref, t1_ref, adjS_ref, bond2_ref, uv_ref,
               bembT_ref, WbdBD_ref, WbbBD_ref, bbT_ref,
               out_ref):
    lane = jax.lax.broadcasted_iota(jnp.int32, (RCHUNK, N), 1)
    oh0 = [(t0_ref[:, jj:jj + 1] == lane).astype(jnp.float32) for jj in range(8)]
    oh1 = [(t1_ref[:, jj:jj + 1] == lane).astype(jnp.float32) for jj in range(8)]
    q = _lane_expand_mat()
    bembT = bembT_ref[...]
    WbdBD = WbdBD_ref[...]
    WbbBD = WbbBD_ref[...]
    bbT = bbT_ref[...]
    for b in range(B):
        u = uv_ref[b, :, :BOND_IN]
        v = uv_ref[b, :, BOND_IN:]
        g = jnp.concatenate(
            [jnp.dot(oh0[jj], u, preferred_element_type=jnp.float32)
             + jnp.dot(oh1[jj], v, preferred_element_type=jnp.float32)
             for jj in range(8)], axis=1)
        diatom = jnp.tanh(g + bembT)
        adjx = jnp.dot(adjS_ref[b], q,
                       preferred_element_type=jnp.float32)
        out_ref[b] = _softplus(
            jnp.dot(diatom, WbdBD, preferred_element_type=jnp.float32)
            + jnp.dot(bond2_ref[b], WbbBD, preferred_element_type=jnp.float32)
            + bbT) * adjx


def kernel(atom, bond, adj_matrix, adj_matrix_tuple,
           W_type_A, b_type_A, W_type_B, b_type_B,
           W_self, W_nbr, W_ab, b_atom,
           W_emb, b_emb, W_bd, W_bb, b_bond):
    f32 = jnp.float32
    bond2 = bond.reshape(B, RB, 128)
    adjS = adj_matrix.reshape(B, RB, 8)
    t0m = adj_matrix_tuple[:, 0].reshape(RB, 8)
    t1m = adj_matrix_tuple[:, 1].reshape(RB, 8)
    eye8 = jnp.asarray(np.eye(8, dtype=np.float32))
    WbdBD = jnp.kron(eye8, W_bd)
    WbbBD = jnp.kron(eye8, W_bb)
    bembT = jnp.tile(b_emb, 8)
    bbT = jnp.tile(b_bond, 8)

    full = lambda shape: pl.BlockSpec(shape, lambda s: (0,) * len(shape))

    atom_update, uv = pl.pallas_call(
        _atom_body,
        grid=(B,),
        in_specs=[
            pl.BlockSpec((1, N, ATOM_RAW), lambda b: (b, 0, 0)),
            pl.BlockSpec((1, N, N), lambda b: (b, 0, 0)),
            pl.BlockSpec((1, RB, 128), lambda b: (b, 0, 0)),
            pl.BlockSpec((1, RB, 8), lambda b: (b, 0, 0)),
            full((ATOM_RAW, TYPE_OUT)), full((TYPE_OUT,)),
            full((ATOM_RAW, TYPE_OUT)), full((TYPE_OUT,)),
            full((TYPE_OUT, ATOM_OUT)), full((TYPE_OUT, ATOM_OUT)),
            full((BOND_IN, ATOM_OUT)), full((ATOM_OUT,)),
            full((2 * ATOM_OUT, BOND_IN)),
        ],
        out_specs=[
            pl.BlockSpec((1, N, ATOM_OUT), lambda b: (b, 0, 0)),
            pl.BlockSpec((1, N, 2 * BOND_IN), lambda b: (b, 0, 0)),
        ],
        out_shape=[
            jax.ShapeDtypeStruct((B, N, ATOM_OUT), f32),
            jax.ShapeDtypeStruct((B, N, 2 * BOND_IN), f32),
        ],
    )(atom, adj_matrix, bond2, adjS,
      W_type_A, b_type_A, W_type_B, b_type_B,
      W_self, W_nbr, W_ab, b_atom, W_emb)

    bond_out2 = pl.pallas_call(
        _bond_body,
        grid=(NSTEPS,),
        in_specs=[
            pl.BlockSpec((RCHUNK, 8), lambda s: (s, 0)),
            pl.BlockSpec((RCHUNK, 8), lambda s: (s, 0)),
            pl.BlockSpec((B, RCHUNK, 8), lambda s: (0, s, 0)),
            pl.BlockSpec((B, RCHUNK, 128), lambda s: (0, s, 0)),
            full((B, N, 2 * BOND_IN)),
            full((128,)), full((128, 128)), full((128, 128)), full((128,)),
        ],
        out_specs=pl.BlockSpec((B, RCHUNK, 128), lambda s: (0, s, 0)),
        out_shape=jax.ShapeDtypeStruct((B, RB, 128), f32),
    )(t0m, t1m, adjS, bond2, uv, bembT, WbdBD, WbbBD, bbT)

    return (atom_update, bond_out2.reshape(B, N, N, BOND_OUT))


# speedup vs baseline: 4.7990x; 4.7990x over previous
import jax
import jax.numpy as jnp
import numpy as np
from jax.experimental import pallas as pl
from jax.experimental.pallas import tpu as pltpu

B = 8
N = 128
P = N * N
ATOM_RAW = 64
TYPE_OUT = 25
ATOM_OUT = 64
BOND_IN = 16
BOND_OUT = 16

CI = 16
NSTEPS = N // CI

_LOG1P_C = (-0.017208061121415857, 0.08172680837598248, -0.18878267362193674,
            0.31459053537151066, -0.49697791116778994, 0.9997924357286251,
            3.507552053038217e-06)


def _softplus(x):
    t = jnp.exp(-jnp.abs(x))
    p = _LOG1P_C[0]
    for c in _LOG1P_C[1:]:
        p = p * t + c
    return jnp.maximum(x, 0.0) + p


def _dg(a, b, dims):
    return jax.lax.dot_general(a, b, (dims, ((), ())),
                               preferred_element_type=jnp.float32)


def _body(atomT_ref, adjA_ref, bondTb_ref, t0_ref, t1_ref, adjB_ref,
          bondTf_ref,
          WA_ref, bA_ref, WB_ref, bB_ref,
          Ws_ref, Wn_ref, Wab_ref, ba_ref, Wemb_ref, bemb2_ref,
          BDT_ref, BBT_ref, bbM_ref,
          atomT_out_ref, outT_ref, uv_ref):
    s = pl.program_id(0)

    @pl.when(s < B)
    def _atom_phase():
        aT = atomT_ref[0]
        adj = adjA_ref[0]
        bondT = bondTb_ref[0]

        tA = jnp.tanh(_dg(WA_ref[...], aT, ((0,), (0,))) + bA_ref[...])
        tB = jnp.tanh(_dg(WB_ref[...], aT, ((0,), (0,))) + bB_ref[...])
        lane = jax.lax.broadcasted_iota(jnp.int32, (TYPE_OUT, N), 1)
        atom_tT = jnp.where(lane < N // 2, tA, tB)

        nbrT = _dg(atom_tT, adj, ((1,), (1,)))
        agg = jnp.sum(bondT * adj[:, None, :], axis=2)

        auT = _softplus(_dg(Ws_ref[...], atom_tT, ((0,), (0,)))
                        + _dg(Wn_ref[...], nbrT, ((0,), (0,)))
                        + _dg(Wab_ref[...], agg, ((0,), (1,)))
                        + ba_ref[...])
        atomT_out_ref[0] = auT
        Wemb = Wemb_ref[...]
        ut = _dg(Wemb[:ATOM_OUT], auT, ((0,), (0,))) + bemb2_ref[...]
        vt = _dg(Wemb[ATOM_OUT:], auT, ((0,), (0,)))
        uv_ref[pl.ds(BOND_IN * s, BOND_IN), 0:N] = ut
        uv_ref[pl.ds(BOND_IN * s, BOND_IN), N:2 * N] = vt

    @pl.when(s >= B)
    def _bond_phase():
        iota_n = jax.lax.broadcasted_iota(jnp.int32, (N, N), 0)
        uv2 = uv_ref[...]
        BDT = BDT_ref[...]
        BBT = BBT_ref[...]
        bbM = bbM_ref[...]
        for il in range(CI):
            r0 = t0_ref[il:il + 1, :]
            r1 = t1_ref[il:il + 1, :]
            oh0 = (iota_n == r0).astype(jnp.float32)
            oh1 = (iota_n == r1).astype(jnp.float32)
            oh2 = jnp.concatenate([oh0, oh1], axis=0)
            d = jnp.dot(uv2, oh2, preferred_element_type=jnp.float32)
            diatom = jnp.tanh(d)
            bcat = jnp.concatenate([bondTf_ref[b, il] for b in range(B)],
                                   axis=0)
            sp = _softplus(
                jnp.dot(BDT, diatom, preferred_element_type=jnp.float32)
                + jnp.dot(BBT, bcat, preferred_element_type=jnp.float32)
                + bbM)
            mask = jnp.concatenate(
                [jnp.broadcast_to(adjB_ref[b, il:il + 1, :], (BOND_OUT, N))
                 for b in range(B)], axis=0)
            outv = sp * mask
            for b in range(B):
                outT_ref[b, il] = outv[BOND_OUT * b:BOND_OUT * (b + 1), :]


def kernel(atom, bond, adj_matrix, adj_matrix_tuple,
           W_type_A, b_type_A, W_type_B, b_type_B,
           W_self, W_nbr, W_ab, b_atom,
           W_emb, b_emb, W_bd, W_bb, b_bond):
    f32 = jnp.float32
    atomT = atom.transpose(0, 2, 1)
    bondT = bond.transpose(0, 1, 3, 2)
    tupT = adj_matrix_tuple.transpose(1, 0)
    t0m = tupT[0].reshape(N, N)
    t1m = tupT[1].reshape(N, N)
    eye8 = jnp.asarray(np.eye(B, dtype=np.float32))
    BDT = jnp.kron(eye8, W_bd.T)
    BBT = jnp.kron(eye8, W_bb.T)
    bbM = jnp.broadcast_to(jnp.tile(b_bond, B)[:, None], (128, N))
    bA2 = jnp.broadcast_to(b_type_A[:, None], (TYPE_OUT, N))
    bB2 = jnp.broadcast_to(b_type_B[:, None], (TYPE_OUT, N))
    baM = jnp.broadcast_to(b_atom[:, None], (ATOM_OUT, N))
    bemb2 = jnp.broadcast_to(b_emb[:, None], (BOND_IN, N))

    full = lambda shape: pl.BlockSpec(shape, lambda s: (0,) * len(shape))
    bidx = lambda s: jnp.minimum(s, B - 1)
    cidx = lambda s: jnp.maximum(s - B, 0)

    atomT_out, outT = pl.pallas_call(
        _body,
        grid=(B + NSTEPS,),
        in_specs=[
            pl.BlockSpec((1, ATOM_RAW, N), lambda s: (bidx(s), 0, 0)),
            pl.BlockSpec((1, N, N), lambda s: (bidx(s), 0, 0)),
            pl.BlockSpec((1, N, BOND_IN, N), lambda s: (bidx(s), 0, 0, 0)),
            pl.BlockSpec((CI, N), lambda s: (cidx(s), 0)),
            pl.BlockSpec((CI, N), lambda s: (cidx(s), 0)),
            pl.BlockSpec((B, CI, N), lambda s: (0, cidx(s), 0)),
            pl.BlockSpec((B, CI, BOND_IN, N), lambda s: (0, cidx(s), 0, 0)),
            full((ATOM_RAW, TYPE_OUT)), full((TYPE_OUT, N)),
            full((ATOM_RAW, TYPE_OUT)), full((TYPE_OUT, N)),
            full((TYPE_OUT, ATOM_OUT)), full((TYPE_OUT, ATOM_OUT)),
            full((BOND_IN, ATOM_OUT)), full((ATOM_OUT, N)),
            full((2 * ATOM_OUT, BOND_IN)), full((BOND_IN, N)),
            full((128, 128)), full((128, 128)), full((128, N)),
        ],
        out_specs=[
            pl.BlockSpec((1, ATOM_OUT, N), lambda s: (bidx(s), 0, 0)),
            pl.BlockSpec((B, CI, BOND_OUT, N), lambda s: (0, cidx(s), 0, 0)),
        ],
        out_shape=[
            jax.ShapeDtypeStruct((B, ATOM_OUT, N), f32),
            jax.ShapeDtypeStruct((B, N, BOND_OUT, N), f32),
        ],
        scratch_shapes=[pltpu.VMEM((B * BOND_IN, 2 * N), f32)],
    )(atomT, adj_matrix, bondT, t0m, t1m, adj_matrix, bondT,
      W_type_A, bA2, W_type_B, bB2,
      W_self, W_nbr, W_ab, baM, W_emb, bemb2,
      BDT, BBT, bbM)

    return (atomT_out.transpose(0, 2, 1), outT.transpose(0, 1, 3, 2))
